# baseline (speedup 1.0000x reference)
import functools

import jax
import jax.numpy as jnp
from jax import lax
from jax.experimental import pallas as pl
from jax.experimental.pallas import tpu as pltpu
from jax.experimental.pallas import tpu_sc as plsc

B = 16384
F = 26
V = 100000
D = 16

_INFO = plsc.get_sparse_core_info()
NC = _INFO.num_cores
NS = _INFO.num_subcores
L = _INFO.num_lanes
NW = NC * NS

VB = 1408
VMAIN = V - (V % 128)
NSLAB = F * (VMAIN // VB)
WPF = 12504
VPAD = WPF * 128 // D
TAILW = F * (V % 128) * D // 128

RW = B // NW
RC = 128
NCH = RW // RC
CN = RC * F

_mesh = plsc.VectorSubcoreMesh(core_axis_name="c", subcore_axis_name="s")


@functools.partial(
    pl.kernel,
    mesh=_mesh,
    out_type=jax.ShapeDtypeStruct((F, WPF, 128), jnp.float32),
    scratch_types=[
        pltpu.VMEM((D, VB), jnp.float32),
        pltpu.VMEM((VB * D // 128, 128), jnp.float32),
        pltpu.VMEM((8, 128), jnp.float32),
    ],
    compiler_params=pltpu.CompilerParams(
        use_tc_tiling_on_sc=True, needs_layout_passes=False
    ),
)
def _transpose_table(tdv_hbm, tail_hbm, w_hbm, slab_v, tslab_v, tail_v):
    wid = lax.axis_index("s") * NC + lax.axis_index("c")
    lane = lax.iota(jnp.int32, L)
    src_cols = [(lane + k) % L for k in range(L)]
    dst_rows = [((lane + k) % L * D + lane) // 128 for k in range(L)]
    dst_cols = [((lane + k) % L * D + lane) % 128 for k in range(L)]

    def slab_body(k, _):
        sid = wid + NW * k

        @pl.when(sid < NSLAB)
        def _():
            fi = sid // (VMAIN // VB)
            v0 = pl.multiple_of((sid % (VMAIN // VB)) * VB, 128)
            pltpu.sync_copy(tdv_hbm.at[fi, :, pl.ds(v0, VB)], slab_v)

            def chunk_body(c, _):
                c16 = c * L
                for k in range(L):
                    x = plsc.load_gather(slab_v, [lane, src_cols[k] + c16])
                    plsc.store_scatter(
                        tslab_v, [dst_rows[k] + c * 2, dst_cols[k]], x
                    )
                return ()

            lax.fori_loop(0, VB // L, chunk_body, ())

            wr0 = pl.multiple_of(v0 * D // 128, 8)
            pltpu.sync_copy(tslab_v, w_hbm.at[fi, pl.ds(wr0, VB * D // 128)])
        return ()

    lax.fori_loop(0, (NSLAB + NW - 1) // NW, slab_body, ())

    @pl.when(wid == 0)
    def _():
        def tail_body(fi, _):
            pltpu.sync_copy(tail_hbm.at[fi], tail_v)
            pltpu.sync_copy(tail_v, w_hbm.at[fi, pl.ds(VMAIN * D // 128, 8)])
            return ()

        lax.fori_loop(0, F, tail_body, ())


@functools.partial(
    pl.kernel,
    mesh=_mesh,
    out_type=jax.ShapeDtypeStruct((B * F, D), jnp.float32),
    scratch_types=[
        pltpu.VMEM((RC, F), jnp.int32),
        pltpu.VMEM((CN,), jnp.int32),
        pltpu.VMEM((CN, D), jnp.float32),
        pltpu.SemaphoreType.DMA,
    ],
    compiler_params=pltpu.CompilerParams(
        use_tc_tiling_on_sc=False, needs_layout_passes=False
    ),
)
def _gather_rows(table_hbm, idx_hbm, out_hbm, idx_v, ids_v, rows_v, sem):
    wid = lax.axis_index("s") * NC + lax.axis_index("c")
    row0 = wid * RW

    off_lo = lax.iota(jnp.int32, L) * VPAD
    off_hi = (lax.iota(jnp.int32, L) + (F - L)) * VPAD

    def chunk_body(i, _):
        b0 = row0 + i * RC
        pltpu.sync_copy(idx_hbm.at[pl.ds(b0, RC), :], idx_v)

        def row_body(r, _):
            ids_v[pl.ds(r * F, L)] = idx_v[r, pl.ds(0, L)] + off_lo
            ids_v[pl.ds(r * F + (F - L), L)] = idx_v[r, pl.ds(F - L, L)] + off_hi
            return ()

        lax.fori_loop(0, RC, row_body, ())

        pltpu.async_copy(table_hbm.at[ids_v], rows_v, sem).wait()
        pltpu.sync_copy(rows_v, out_hbm.at[pl.ds(b0 * F, CN)])
        return ()

    lax.fori_loop(0, NCH, chunk_body, ())


def kernel(indices, tables):
    tdv = tables.transpose(0, 2, 1)
    tail = jnp.concatenate(
        [
            tables[:, VMAIN:, :].reshape(F, 4, 128),
            jnp.zeros((F, 4, 128), jnp.float32),
        ],
        axis=1,
    )
    w = _transpose_table(tdv, tail)
    out = _gather_rows(w.reshape(F * VPAD, D), indices)
    return out.reshape(B, F * D)


# speedup vs baseline: 1.7503x; 1.7503x over previous
import functools

import jax
import jax.numpy as jnp
from jax import lax
from jax.experimental import pallas as pl
from jax.experimental.pallas import tpu as pltpu
from jax.experimental.pallas import tpu_sc as plsc

B = 16384
F = 26
V = 100000
D = 16

_INFO = plsc.get_sparse_core_info()
NC = _INFO.num_cores
NS = _INFO.num_subcores
L = _INFO.num_lanes
NW = NC * NS

VB = 1408
VMAIN = V - (V % 128)
NSLAB = F * (VMAIN // VB)
WPF = 12504
VPAD = WPF * 128 // D
TAILW = F * (V % 128) * D // 128

RW = B // NW
RC = 128
NCH = RW // RC
CN = RC * F

_mesh = plsc.VectorSubcoreMesh(core_axis_name="c", subcore_axis_name="s")


@functools.partial(
    pl.kernel,
    mesh=_mesh,
    out_type=jax.ShapeDtypeStruct((F, WPF, 128), jnp.float32),
    scratch_types=[
        pltpu.VMEM((D, VB), jnp.float32),
        pltpu.VMEM((VB * D // 128, 128), jnp.float32),
        pltpu.VMEM((8, 128), jnp.float32),
    ],
    compiler_params=pltpu.CompilerParams(
        use_tc_tiling_on_sc=True, needs_layout_passes=False
    ),
)
def _transpose_table(tdv_hbm, tail_hbm, w_hbm, slab_v, tslab_v, tail_v):
    wid = lax.axis_index("s") * NC + lax.axis_index("c")
    lane = lax.iota(jnp.int32, L)
    row_base = lane // 8
    col_bases = [(lane % 8) * D + d for d in range(D)]

    def slab_body(k, _):
        sid = wid + NW * k

        @pl.when(sid < NSLAB)
        def _():
            fi = sid // (VMAIN // VB)
            v0 = pl.multiple_of((sid % (VMAIN // VB)) * VB, 128)
            pltpu.sync_copy(tdv_hbm.at[fi, :, pl.ds(v0, VB)], slab_v)

            @plsc.parallel_loop(0, VB // L, 1)
            def chunk_body(c):
                rows = row_base + c * 2
                c16 = c * L
                for d in range(D):
                    x = slab_v[d, pl.ds(c16, L)]
                    plsc.store_scatter(tslab_v, [rows, col_bases[d]], x)

            wr0 = pl.multiple_of(v0 * D // 128, 8)
            pltpu.sync_copy(tslab_v, w_hbm.at[fi, pl.ds(wr0, VB * D // 128)])
        return ()

    lax.fori_loop(0, (NSLAB + NW - 1) // NW, slab_body, ())

    @pl.when(wid == 0)
    def _():
        def tail_body(fi, _):
            pltpu.sync_copy(tail_hbm.at[fi], tail_v)
            pltpu.sync_copy(tail_v, w_hbm.at[fi, pl.ds(VMAIN * D // 128, 8)])
            return ()

        lax.fori_loop(0, F, tail_body, ())


@functools.partial(
    pl.kernel,
    mesh=_mesh,
    out_type=jax.ShapeDtypeStruct((B * F, D), jnp.float32),
    scratch_types=[
        pltpu.VMEM((RC, F), jnp.int32),
        pltpu.VMEM((CN,), jnp.int32),
        pltpu.VMEM((CN, D), jnp.float32),
        pltpu.SemaphoreType.DMA,
    ],
    compiler_params=pltpu.CompilerParams(
        use_tc_tiling_on_sc=False, needs_layout_passes=False
    ),
)
def _gather_rows(table_hbm, idx_hbm, out_hbm, idx_v, ids_v, rows_v, sem):
    wid = lax.axis_index("s") * NC + lax.axis_index("c")
    row0 = wid * RW

    off_lo = lax.iota(jnp.int32, L) * VPAD
    off_hi = (lax.iota(jnp.int32, L) + (F - L)) * VPAD

    def chunk_body(i, _):
        b0 = row0 + i * RC
        pltpu.sync_copy(idx_hbm.at[pl.ds(b0, RC), :], idx_v)

        def row_body(r, _):
            ids_v[pl.ds(r * F, L)] = idx_v[r, pl.ds(0, L)] + off_lo
            ids_v[pl.ds(r * F + (F - L), L)] = idx_v[r, pl.ds(F - L, L)] + off_hi
            return ()

        lax.fori_loop(0, RC, row_body, ())

        pltpu.async_copy(table_hbm.at[ids_v], rows_v, sem).wait()
        pltpu.sync_copy(rows_v, out_hbm.at[pl.ds(b0 * F, CN)])
        return ()

    lax.fori_loop(0, NCH, chunk_body, ())


def kernel(indices, tables):
    tdv = tables.transpose(0, 2, 1)
    tail = jnp.concatenate(
        [
            tables[:, VMAIN:, :].reshape(F, 4, 128),
            jnp.zeros((F, 4, 128), jnp.float32),
        ],
        axis=1,
    )
    w = _transpose_table(tdv, tail)
    out = _gather_rows(w.reshape(F * VPAD, D), indices)
    return out.reshape(B, F * D)


# speedup vs baseline: 1.9118x; 1.0922x over previous
import functools

import jax
import jax.numpy as jnp
from jax import lax
from jax.experimental import pallas as pl
from jax.experimental.pallas import tpu as pltpu
from jax.experimental.pallas import tpu_sc as plsc

B = 16384
F = 26
V = 100000
D = 16

_INFO = plsc.get_sparse_core_info()
NC = _INFO.num_cores
NS = _INFO.num_subcores
L = _INFO.num_lanes
NW = NC * NS

VB = 1408
VMAIN = V - (V % 128)
NSLAB = F * (VMAIN // VB)
WPF = 12504
VPAD = WPF * 128 // D
TAILW = F * (V % 128) * D // 128

RW = B // NW
RC = 128
NCH = RW // RC
CN = RC * F

_mesh = plsc.VectorSubcoreMesh(core_axis_name="c", subcore_axis_name="s")


@functools.partial(
    pl.kernel,
    mesh=_mesh,
    out_type=jax.ShapeDtypeStruct((F, WPF, 128), jnp.float32),
    scratch_types=[
        pltpu.VMEM((D, VB), jnp.float32),
        pltpu.VMEM((VB * D // 128, 128), jnp.float32),
        pltpu.VMEM((VB * D // 128, 128), jnp.float32),
        pltpu.VMEM((8, 128), jnp.float32),
        pltpu.SemaphoreType.DMA,
        pltpu.SemaphoreType.DMA,
    ],
    compiler_params=pltpu.CompilerParams(
        use_tc_tiling_on_sc=True, needs_layout_passes=False
    ),
)
def _transpose_table(
    tdv_hbm, tail_hbm, w_hbm, slab_v, tslab0_v, tslab1_v, tail_v, sem0, sem1
):
    wid = lax.axis_index("s") * NC + lax.axis_index("c")
    lane = lax.iota(jnp.int32, L)
    row_base = lane // 8
    col_bases = [(lane % 8) * D + d for d in range(D)]

    tslabs = (tslab0_v, tslab1_v)
    sems = (sem0, sem1)
    wr_len = VB * D // 128

    def do_slab(k, j, drain):
        sid = wid + NW * k
        tslab_v, sem = tslabs[j], sems[j]

        @pl.when(sid < NSLAB)
        def _():
            fi = sid // (VMAIN // VB)
            v0 = pl.multiple_of((sid % (VMAIN // VB)) * VB, 128)
            pltpu.sync_copy(tdv_hbm.at[fi, :, pl.ds(v0, VB)], slab_v)

            if drain:
                pltpu.make_async_copy(
                    w_hbm.at[0, pl.ds(0, wr_len)], tslab_v, sem
                ).wait()

            @plsc.parallel_loop(0, VB // L, 1)
            def chunk_body(c):
                rows = row_base + c * 2
                c16 = c * L
                for d in range(D):
                    x = slab_v[d, pl.ds(c16, L)]
                    plsc.store_scatter(tslab_v, [rows, col_bases[d]], x)

            wr0 = pl.multiple_of(v0 * D // 128, 8)
            pltpu.async_copy(tslab_v, w_hbm.at[fi, pl.ds(wr0, wr_len)], sem)

    do_slab(0, 0, False)
    do_slab(1, 1, False)

    def slab_pair_body(k2, _):
        for j in range(2):
            do_slab(k2 * 2 + j, j, True)
        return ()

    npairs = ((NSLAB + NW - 1) // NW + 1) // 2
    lax.fori_loop(1, npairs, slab_pair_body, ())

    for j in range(2):
        pltpu.make_async_copy(
            w_hbm.at[0, pl.ds(0, wr_len)], tslabs[j], sems[j]
        ).wait()

    @pl.when(wid == 0)
    def _():
        def tail_body(fi, _):
            pltpu.sync_copy(tail_hbm.at[fi], tail_v)
            pltpu.sync_copy(tail_v, w_hbm.at[fi, pl.ds(VMAIN * D // 128, 8)])
            return ()

        lax.fori_loop(0, F, tail_body, ())


@functools.partial(
    pl.kernel,
    mesh=_mesh,
    out_type=jax.ShapeDtypeStruct((B * F, D), jnp.float32),
    scratch_types=[
        pltpu.VMEM((RC, F), jnp.int32),
        pltpu.VMEM((CN,), jnp.int32),
        pltpu.VMEM((CN, D), jnp.float32),
        pltpu.SemaphoreType.DMA,
    ],
    compiler_params=pltpu.CompilerParams(
        use_tc_tiling_on_sc=False, needs_layout_passes=False
    ),
)
def _gather_rows(table_hbm, idx_hbm, out_hbm, idx_v, ids_v, rows_v, sem):
    wid = lax.axis_index("s") * NC + lax.axis_index("c")
    row0 = wid * RW

    off_lo = lax.iota(jnp.int32, L) * VPAD
    off_hi = (lax.iota(jnp.int32, L) + (F - L)) * VPAD

    def chunk_body(i, _):
        b0 = row0 + i * RC
        pltpu.sync_copy(idx_hbm.at[pl.ds(b0, RC), :], idx_v)

        def row_body(r, _):
            ids_v[pl.ds(r * F, L)] = idx_v[r, pl.ds(0, L)] + off_lo
            ids_v[pl.ds(r * F + (F - L), L)] = idx_v[r, pl.ds(F - L, L)] + off_hi
            return ()

        lax.fori_loop(0, RC, row_body, ())

        pltpu.async_copy(table_hbm.at[ids_v], rows_v, sem).wait()
        pltpu.sync_copy(rows_v, out_hbm.at[pl.ds(b0 * F, CN)])
        return ()

    lax.fori_loop(0, NCH, chunk_body, ())


def kernel(indices, tables):
    tdv = tables.transpose(0, 2, 1)
    tail = jnp.concatenate(
        [
            tables[:, VMAIN:, :].reshape(F, 4, 128),
            jnp.zeros((F, 4, 128), jnp.float32),
        ],
        axis=1,
    )
    w = _transpose_table(tdv, tail)
    out = _gather_rows(w.reshape(F * VPAD, D), indices)
    return out.reshape(B, F * D)


# speedup vs baseline: 2.3918x; 1.2511x over previous
import functools

import jax
import jax.numpy as jnp
from jax import lax
from jax.experimental import pallas as pl
from jax.experimental.pallas import tpu as pltpu
from jax.experimental.pallas import tpu_sc as plsc

B = 16384
F = 26
V = 100000
D = 16

_INFO = plsc.get_sparse_core_info()
NC = _INFO.num_cores
NS = _INFO.num_subcores
L = _INFO.num_lanes
NW = NC * NS

VB = 1408
VMAIN = V - (V % 128)
NSLAB = F * (VMAIN // VB)
WPF = 12504
VPAD = WPF * 128 // D
TAILW = F * (V % 128) * D // 128

RW = B // NW
RC = 128
NCH = RW // RC
CN = RC * F

_mesh = plsc.VectorSubcoreMesh(core_axis_name="c", subcore_axis_name="s")


@functools.partial(
    pl.kernel,
    mesh=_mesh,
    out_type=jax.ShapeDtypeStruct((F, WPF, 128), jnp.float32),
    scratch_types=[
        pltpu.VMEM((D, VB), jnp.float32),
        pltpu.VMEM((D, VB), jnp.float32),
        pltpu.VMEM((VB * D // 128, 128), jnp.float32),
        pltpu.VMEM((VB * D // 128, 128), jnp.float32),
        pltpu.VMEM((8, 128), jnp.float32),
        pltpu.SemaphoreType.DMA,
        pltpu.SemaphoreType.DMA,
        pltpu.SemaphoreType.DMA,
        pltpu.SemaphoreType.DMA,
    ],
    compiler_params=pltpu.CompilerParams(
        use_tc_tiling_on_sc=True, needs_layout_passes=False
    ),
)
def _transpose_table(
    tdv_hbm, tail_hbm, w_hbm,
    slab0_v, slab1_v, tslab0_v, tslab1_v, tail_v,
    isem0, isem1, osem0, osem1,
):
    wid = lax.axis_index("s") * NC + lax.axis_index("c")
    lane = lax.iota(jnp.int32, L)
    row_base = lane // 8
    col_bases = [(lane % 8) * D + d for d in range(D)]

    slabs = (slab0_v, slab1_v)
    tslabs = (tslab0_v, tslab1_v)
    isems = (isem0, isem1)
    osems = (osem0, osem1)
    wr_len = VB * D // 128

    def issue_in(k, j):
        sid = wid + NW * k

        @pl.when(sid < NSLAB)
        def _():
            fi = sid // (VMAIN // VB)
            v0 = pl.multiple_of((sid % (VMAIN // VB)) * VB, 128)
            pltpu.async_copy(tdv_hbm.at[fi, :, pl.ds(v0, VB)], slabs[j], isems[j])

    def do_slab(k, j, drain_out):
        sid = wid + NW * k
        slab_v, tslab_v = slabs[j], tslabs[j]

        @pl.when(sid < NSLAB)
        def _():
            issue_in(k + 1, 1 - j)

            pltpu.make_async_copy(
                tdv_hbm.at[0, :, pl.ds(0, VB)], slab_v, isems[j]
            ).wait()

            if drain_out:
                pltpu.make_async_copy(
                    w_hbm.at[0, pl.ds(0, wr_len)], tslab_v, osems[j]
                ).wait()

            @plsc.parallel_loop(0, VB // L, 1)
            def chunk_body(c):
                rows = row_base + c * 2
                c16 = c * L
                for d in range(D):
                    x = slab_v[d, pl.ds(c16, L)]
                    plsc.store_scatter(tslab_v, [rows, col_bases[d]], x)

            fi = sid // (VMAIN // VB)
            v0 = pl.multiple_of((sid % (VMAIN // VB)) * VB, 128)
            wr0 = pl.multiple_of(v0 * D // 128, 8)
            pltpu.async_copy(tslab_v, w_hbm.at[fi, pl.ds(wr0, wr_len)], osems[j])

    issue_in(0, 0)
    do_slab(0, 0, False)
    do_slab(1, 1, False)

    def slab_pair_body(k2, _):
        for j in range(2):
            do_slab(k2 * 2 + j, j, True)
        return ()

    npairs = ((NSLAB + NW - 1) // NW + 1) // 2
    lax.fori_loop(1, npairs, slab_pair_body, ())

    for j in range(2):
        pltpu.make_async_copy(
            w_hbm.at[0, pl.ds(0, wr_len)], tslabs[j], osems[j]
        ).wait()

    @pl.when(wid == 0)
    def _():
        def tail_body(fi, _):
            pltpu.sync_copy(tail_hbm.at[fi], tail_v)
            pltpu.sync_copy(tail_v, w_hbm.at[fi, pl.ds(VMAIN * D // 128, 8)])
            return ()

        lax.fori_loop(0, F, tail_body, ())


@functools.partial(
    pl.kernel,
    mesh=_mesh,
    out_type=jax.ShapeDtypeStruct((B * F, D), jnp.float32),
    scratch_types=[
        pltpu.VMEM((RC, F), jnp.int32),
        pltpu.VMEM((CN,), jnp.int32),
        pltpu.VMEM((CN, D), jnp.float32),
        pltpu.SemaphoreType.DMA,
    ],
    compiler_params=pltpu.CompilerParams(
        use_tc_tiling_on_sc=False, needs_layout_passes=False
    ),
)
def _gather_rows(table_hbm, idx_hbm, out_hbm, idx_v, ids_v, rows_v, sem):
    wid = lax.axis_index("s") * NC + lax.axis_index("c")
    row0 = wid * RW

    off_lo = lax.iota(jnp.int32, L) * VPAD
    off_hi = (lax.iota(jnp.int32, L) + (F - L)) * VPAD

    def chunk_body(i, _):
        b0 = row0 + i * RC
        pltpu.sync_copy(idx_hbm.at[pl.ds(b0, RC), :], idx_v)

        def row_body(r, _):
            ids_v[pl.ds(r * F, L)] = idx_v[r, pl.ds(0, L)] + off_lo
            ids_v[pl.ds(r * F + (F - L), L)] = idx_v[r, pl.ds(F - L, L)] + off_hi
            return ()

        lax.fori_loop(0, RC, row_body, ())

        pltpu.async_copy(table_hbm.at[ids_v], rows_v, sem).wait()
        pltpu.sync_copy(rows_v, out_hbm.at[pl.ds(b0 * F, CN)])
        return ()

    lax.fori_loop(0, NCH, chunk_body, ())


def kernel(indices, tables):
    tdv = tables.transpose(0, 2, 1)
    tail = jnp.concatenate(
        [
            tables[:, VMAIN:, :].reshape(F, 4, 128),
            jnp.zeros((F, 4, 128), jnp.float32),
        ],
        axis=1,
    )
    w = _transpose_table(tdv, tail)
    out = _gather_rows(w.reshape(F * VPAD, D), indices)
    return out.reshape(B, F * D)


# speedup vs baseline: 2.3935x; 1.0007x over previous
import functools

import jax
import jax.numpy as jnp
from jax import lax
from jax.experimental import pallas as pl
from jax.experimental.pallas import tpu as pltpu
from jax.experimental.pallas import tpu_sc as plsc

B = 16384
F = 26
V = 100000
D = 16

_INFO = plsc.get_sparse_core_info()
NC = _INFO.num_cores
NS = _INFO.num_subcores
L = _INFO.num_lanes
NW = NC * NS

VB = 1408
VMAIN = V - (V % 128)
NSLAB = F * (VMAIN // VB)
WPF = 12504
VPAD = WPF * 128 // D

RW = B // NW
RC = 128
NCH = RW // RC
CN = RC * F

_mesh = plsc.VectorSubcoreMesh(core_axis_name="c", subcore_axis_name="s")


@functools.partial(
    pl.kernel,
    mesh=_mesh,
    out_type=jax.ShapeDtypeStruct((F, WPF, 128), jnp.float32),
    scratch_types=[
        pltpu.VMEM((D, VB), jnp.float32),
        pltpu.VMEM((D, VB), jnp.float32),
        pltpu.VMEM((VB * D // 128, 128), jnp.float32),
        pltpu.VMEM((VB * D // 128, 128), jnp.float32),
        pltpu.VMEM((8, 128), jnp.float32),
        pltpu.SemaphoreType.DMA,
        pltpu.SemaphoreType.DMA,
        pltpu.SemaphoreType.DMA,
        pltpu.SemaphoreType.DMA,
    ],
    compiler_params=pltpu.CompilerParams(
        use_tc_tiling_on_sc=True, needs_layout_passes=False
    ),
)
def _transpose_table(
    tdv_hbm, tail_hbm, w_hbm,
    slab0_v, slab1_v, tslab0_v, tslab1_v, tail_v,
    isem0, isem1, osem0, osem1,
):
    wid = lax.axis_index("s") * NC + lax.axis_index("c")
    lane = lax.iota(jnp.int32, L)
    row_base = lane // 8
    col_bases = [(lane % 8) * D + d for d in range(D)]

    slabs = (slab0_v, slab1_v)
    tslabs = (tslab0_v, tslab1_v)
    isems = (isem0, isem1)
    osems = (osem0, osem1)
    wr_len = VB * D // 128

    def issue_in(k, j):
        sid = wid + NW * k

        @pl.when(sid < NSLAB)
        def _():
            fi = sid // (VMAIN // VB)
            v0 = pl.multiple_of((sid % (VMAIN // VB)) * VB, 128)
            pltpu.async_copy(tdv_hbm.at[fi, :, pl.ds(v0, VB)], slabs[j], isems[j])

    def do_slab(k, j, drain_out):
        sid = wid + NW * k
        slab_v, tslab_v = slabs[j], tslabs[j]

        @pl.when(sid < NSLAB)
        def _():
            issue_in(k + 1, 1 - j)

            pltpu.make_async_copy(
                tdv_hbm.at[0, :, pl.ds(0, VB)], slab_v, isems[j]
            ).wait()

            if drain_out:
                pltpu.make_async_copy(
                    w_hbm.at[0, pl.ds(0, wr_len)], tslab_v, osems[j]
                ).wait()

            @plsc.parallel_loop(0, VB // L, 1)
            def chunk_body(c):
                rows = row_base + c * 2
                c16 = c * L
                for d in range(D):
                    x = slab_v[d, pl.ds(c16, L)]
                    plsc.store_scatter(tslab_v, [rows, col_bases[d]], x)

            fi = sid // (VMAIN // VB)
            v0 = pl.multiple_of((sid % (VMAIN // VB)) * VB, 128)
            wr0 = pl.multiple_of(v0 * D // 128, 8)
            pltpu.async_copy(tslab_v, w_hbm.at[fi, pl.ds(wr0, wr_len)], osems[j])

    issue_in(0, 0)
    do_slab(0, 0, False)
    do_slab(1, 1, False)

    def slab_pair_body(k2, _):
        for j in range(2):
            do_slab(k2 * 2 + j, j, True)
        return ()

    npairs = ((NSLAB + NW - 1) // NW + 1) // 2
    lax.fori_loop(1, npairs, slab_pair_body, ())

    for j in range(2):
        pltpu.make_async_copy(
            w_hbm.at[0, pl.ds(0, wr_len)], tslabs[j], osems[j]
        ).wait()

    @pl.when(wid == 0)
    def _():
        def tail_body(fi, _):
            pltpu.sync_copy(tail_hbm.at[fi], tail_v)
            pltpu.sync_copy(tail_v, w_hbm.at[fi, pl.ds(VMAIN * D // 128, 8)])
            return ()

        lax.fori_loop(0, F, tail_body, ())


@functools.partial(
    pl.kernel,
    mesh=_mesh,
    out_type=jax.ShapeDtypeStruct((B * F, D), jnp.float32),
    scratch_types=[
        pltpu.VMEM((RC, F), jnp.int32),
        pltpu.VMEM((CN,), jnp.int32),
        pltpu.VMEM((CN, D), jnp.float32),
        pltpu.SemaphoreType.DMA,
    ],
    compiler_params=pltpu.CompilerParams(
        use_tc_tiling_on_sc=False, needs_layout_passes=False
    ),
)
def _gather_rows(table_hbm, idx_hbm, out_hbm, idx_v, ids_v, rows_v, sem):
    wid = lax.axis_index("s") * NC + lax.axis_index("c")
    row0 = wid * RW

    off_lo = lax.iota(jnp.int32, L) * VPAD
    off_hi = (lax.iota(jnp.int32, L) + (F - L)) * VPAD

    def chunk_body(i, _):
        b0 = row0 + i * RC
        pltpu.sync_copy(idx_hbm.at[pl.ds(b0, RC), :], idx_v)

        def row_body(r, _):
            ids_v[pl.ds(r * F, L)] = idx_v[r, pl.ds(0, L)] + off_lo
            ids_v[pl.ds(r * F + (F - L), L)] = idx_v[r, pl.ds(F - L, L)] + off_hi
            return ()

        lax.fori_loop(0, RC, row_body, ())

        pltpu.async_copy(table_hbm.at[ids_v], rows_v, sem).wait()
        pltpu.sync_copy(rows_v, out_hbm.at[pl.ds(b0 * F, CN)])
        return ()

    lax.fori_loop(0, NCH, chunk_body, ())


def kernel(indices, tables):
    tdv = tables.transpose(0, 2, 1)
    tail = jnp.concatenate(
        [
            tables[:, VMAIN:, :].reshape(F, 4, 128),
            jnp.zeros((F, 4, 128), jnp.float32),
        ],
        axis=1,
    )
    w = _transpose_table(tdv, tail)
    out = _gather_rows(w.reshape(F * VPAD, D), indices)
    return out.reshape(B, F * D)
